# Initial kernel scaffold; baseline (speedup 1.0000x reference)
import jax
import jax.numpy as jnp
from jax.experimental import pallas as pl


def kernel(x, ew0, eb0, ew1, eb1, ew2, eb2, codebooks, dw0, db0, dw1, db1, dw2, db2):
    raise NotImplementedError("write your pallas kernel here")


# speedup vs baseline: 1.0938x; 1.0938x over previous
import functools

import jax
import jax.numpy as jnp
from jax.experimental import pallas as pl
from jax.experimental.pallas import tpu as pltpu


def _bdot(a, b):
    return jnp.dot(a.astype(jnp.bfloat16), b.astype(jnp.bfloat16),
                   preferred_element_type=jnp.float32)


def _fused_body(num_levels, blk,
                x_ref, ew0_ref, eb0_ref, ew1_ref, eb1_ref, ew2_ref, eb2_ref,
                cb_ref, cbT_ref, dw0_ref, db0_ref, dw1_ref, db1_ref,
                dw2_ref, db2_ref,
                dec_ref, r_ref, e_ref, counts_ref, qT_ref):
    f32 = jnp.float32
    K = cb_ref.shape[1]

    @pl.when(pl.program_id(0) == 0)
    def _init_counts():
        counts_ref[...] = jnp.zeros_like(counts_ref)

    x = x_ref[...]
    h = jnp.maximum(_bdot(x, ew0_ref[...]) + eb0_ref[...], 0.0)
    h = jnp.maximum(_bdot(h, ew1_ref[...]) + eb1_ref[...], 0.0)
    z = _bdot(h, ew2_ref[...]) + eb2_ref[...]

    residual = z
    zq = jnp.zeros_like(z)
    for l in range(num_levels):
        cb = cb_ref[l]
        cbT = cbT_ref[l]
        r2 = jnp.sum(residual * residual, axis=1, keepdims=True)
        c2 = jnp.sum(cb * cb, axis=1)[None, :]
        d = (r2
             - 2.0 * jnp.dot(residual.astype(jnp.bfloat16), cbT,
                             preferred_element_type=f32)
             + c2)
        idx = jnp.argmin(d, axis=1).astype(jnp.int32)
        onehot = (jax.lax.broadcasted_iota(jnp.int32, (blk, K), 1)
                  == idx[:, None])
        e_l = jnp.dot(onehot.astype(f32), cb, preferred_element_type=f32,
                      precision=jax.lax.Precision.HIGHEST)
        counts_ref[l, :] += jnp.sum(onehot.astype(jnp.int32), axis=0)
        r_ref[:, l, :] = residual
        e_ref[:, l, :] = e_l
        qT_ref[l, :] = idx
        zq = zq + e_l
        residual = residual - e_l

    h = jnp.maximum(_bdot(zq, dw0_ref[...]) + db0_ref[...], 0.0)
    h = jnp.maximum(_bdot(h, dw1_ref[...]) + db1_ref[...], 0.0)
    dec_ref[...] = _bdot(h, dw2_ref[...]) + db2_ref[...]


def kernel(x, ew0, eb0, ew1, eb1, ew2, eb2, codebooks,
           dw0, db0, dw1, db1, dw2, db2):
    B, D_in = x.shape
    L, K, D = codebooks.shape
    H0 = ew0.shape[1]
    H1 = ew1.shape[1]
    D_out = dw2.shape[1]

    blk = 256 if B % 256 == 0 else B
    grid = (B // blk,)

    cbT = codebooks.transpose(0, 2, 1).astype(jnp.bfloat16)

    def rows(*shape_tail):
        return pl.BlockSpec((blk,) + shape_tail,
                            lambda i: (i,) + (0,) * len(shape_tail))

    def whole(*shape):
        return pl.BlockSpec(shape, lambda i: (0,) * len(shape))

    in_specs = [
        rows(D_in),
        whole(D_in, H0), whole(1, H0),
        whole(H0, H1), whole(1, H1),
        whole(H1, D), whole(1, D),
        whole(L, K, D),
        whole(L, D, K),
        whole(D, H1), whole(1, H1),
        whole(H1, H0), whole(1, H0),
        whole(H0, D_out), whole(1, D_out),
    ]
    out_specs = (
        rows(D_out),
        rows(L, D),
        rows(L, D),
        whole(L, K),
        pl.BlockSpec((L, blk), lambda i: (0, i)),
    )
    out_shape = (
        jax.ShapeDtypeStruct((B, D_out), jnp.float32),
        jax.ShapeDtypeStruct((B, L, D), jnp.float32),
        jax.ShapeDtypeStruct((B, L, D), jnp.float32),
        jax.ShapeDtypeStruct((L, K), jnp.int32),
        jax.ShapeDtypeStruct((L, B), jnp.int32),
    )

    body = functools.partial(_fused_body, L, blk)
    dec, r, e, counts, qT = pl.pallas_call(
        body,
        grid=grid,
        in_specs=in_specs,
        out_specs=out_specs,
        out_shape=out_shape,
        compiler_params=pltpu.CompilerParams(
            dimension_semantics=("arbitrary",),
        ),
    )(x, ew0, eb0.reshape(1, -1), ew1, eb1.reshape(1, -1),
      ew2, eb2.reshape(1, -1), codebooks, cbT,
      dw0, db0.reshape(1, -1), dw1, db1.reshape(1, -1),
      dw2, db2.reshape(1, -1))

    return (dec, r, e, counts, qT.T)


# speedup vs baseline: 1.1182x; 1.0223x over previous
import functools

import jax
import jax.numpy as jnp
from jax.experimental import pallas as pl
from jax.experimental.pallas import tpu as pltpu


def _bdot(a, b):
    return jnp.dot(a.astype(jnp.bfloat16), b.astype(jnp.bfloat16),
                   preferred_element_type=jnp.float32)


def _fused_body(num_levels, blk,
                x_ref, ew0_ref, eb0_ref, ew1_ref, eb1_ref, ew2_ref, eb2_ref,
                cb_ref, cbT_ref, dw0_ref, db0_ref, dw1_ref, db1_ref,
                dw2_ref, db2_ref,
                dec_ref, r_ref, e_ref, counts_ref, qT_ref):
    f32 = jnp.float32
    K = cb_ref.shape[1]

    @pl.when(pl.program_id(0) == 0)
    def _init_counts():
        counts_ref[...] = jnp.zeros_like(counts_ref)

    x = x_ref[...]
    h = jnp.maximum(_bdot(x, ew0_ref[...]) + eb0_ref[...], 0.0)
    h = jnp.maximum(_bdot(h, ew1_ref[...]) + eb1_ref[...], 0.0)
    z = _bdot(h, ew2_ref[...]) + eb2_ref[...]

    residual = z
    zq = jnp.zeros_like(z)
    for l in range(num_levels):
        cb = cb_ref[l]
        cbT = cbT_ref[l]
        r2 = jnp.sum(residual * residual, axis=1, keepdims=True)
        c2 = jnp.sum(cb * cb, axis=1)[None, :]
        d = (r2
             - 2.0 * jnp.dot(residual.astype(jnp.bfloat16), cbT,
                             preferred_element_type=f32)
             + c2)
        idx = jnp.argmin(d, axis=1).astype(jnp.int32)
        onehot = (jax.lax.broadcasted_iota(jnp.int32, (blk, K), 1)
                  == idx[:, None]).astype(f32)
        e_l = jnp.dot(onehot, cb, preferred_element_type=f32,
                      precision=jax.lax.Precision.HIGHEST)
        counts_ref[l, :] += jnp.sum(onehot, axis=0).astype(jnp.int32)
        r_ref[:, l, :] = residual
        e_ref[:, l, :] = e_l
        qT_ref[l, :] = idx
        zq = zq + e_l
        residual = residual - e_l

    h = jnp.maximum(_bdot(zq, dw0_ref[...]) + db0_ref[...], 0.0)
    h = jnp.maximum(_bdot(h, dw1_ref[...]) + db1_ref[...], 0.0)
    dec_ref[...] = _bdot(h, dw2_ref[...]) + db2_ref[...]


def kernel(x, ew0, eb0, ew1, eb1, ew2, eb2, codebooks,
           dw0, db0, dw1, db1, dw2, db2):
    B, D_in = x.shape
    L, K, D = codebooks.shape
    H0 = ew0.shape[1]
    H1 = ew1.shape[1]
    D_out = dw2.shape[1]

    blk = 256 if B % 256 == 0 else B
    grid = (B // blk,)

    cbT = codebooks.transpose(0, 2, 1).astype(jnp.bfloat16)

    def rows(*shape_tail):
        return pl.BlockSpec((blk,) + shape_tail,
                            lambda i: (i,) + (0,) * len(shape_tail))

    def whole(*shape):
        return pl.BlockSpec(shape, lambda i: (0,) * len(shape))

    in_specs = [
        rows(D_in),
        whole(D_in, H0), whole(1, H0),
        whole(H0, H1), whole(1, H1),
        whole(H1, D), whole(1, D),
        whole(L, K, D),
        whole(L, D, K),
        whole(D, H1), whole(1, H1),
        whole(H1, H0), whole(1, H0),
        whole(H0, D_out), whole(1, D_out),
    ]
    out_specs = (
        rows(D_out),
        rows(L, D),
        rows(L, D),
        whole(L, K),
        pl.BlockSpec((L, blk), lambda i: (0, i)),
    )
    out_shape = (
        jax.ShapeDtypeStruct((B, D_out), jnp.float32),
        jax.ShapeDtypeStruct((B, L, D), jnp.float32),
        jax.ShapeDtypeStruct((B, L, D), jnp.float32),
        jax.ShapeDtypeStruct((L, K), jnp.int32),
        jax.ShapeDtypeStruct((L, B), jnp.int32),
    )

    body = functools.partial(_fused_body, L, blk)
    dec, r, e, counts, qT = pl.pallas_call(
        body,
        grid=grid,
        in_specs=in_specs,
        out_specs=out_specs,
        out_shape=out_shape,
        compiler_params=pltpu.CompilerParams(
            dimension_semantics=("arbitrary",),
        ),
    )(x, ew0, eb0.reshape(1, -1), ew1, eb1.reshape(1, -1),
      ew2, eb2.reshape(1, -1), codebooks, cbT,
      dw0, db0.reshape(1, -1), dw1, db1.reshape(1, -1),
      dw2, db2.reshape(1, -1))

    return (dec, r, e, counts, qT.T)
